# baseline (speedup 1.0000x reference)
import functools

import jax
import jax.numpy as jnp
from jax import lax
from jax.experimental import pallas as pl
from jax.experimental.pallas import tpu as pltpu
from jax.experimental.pallas import tpu_sc as plsc

DIM = 64
PDIM = 128
N_HEADS = 26
BATCH = 16384
N_ROWS = BATCH * N_HEADS
N_PAIRS = 2600000 // 2
NC, NS, L = 2, 16, 16
NW = NC * NS
ROWS_PER_W = N_ROWS // NW
CHUNK = 128
N_CHUNKS = ROWS_PER_W // CHUNK
VREGS_PER_CHUNK = CHUNK // L
NBUF = 4
N_GROUPS = N_CHUNKS // NBUF


def _sc_gather(ids_hbm, offs_hbm, table_hbm, out_hbm,
               idx_v, offs_v, rows_v, wb_v, gsem, osem):
    wid = lax.axis_index("s") * NC + lax.axis_index("c")
    pltpu.sync_copy(ids_hbm.at[wid], idx_v)
    pltpu.sync_copy(offs_hbm, offs_v)
    out_base = wid * ROWS_PER_W

    def prep_indices(j):
        for k in range(VREGS_PER_CHUNK):
            sl = pl.ds(k * L, L)
            shifted = idx_v[j, sl] + offs_v[j, sl]
            offs_v[j, sl] = (shifted & 1) * DIM
            idx_v[j, sl] = shifted >> 1

    def gather(j, b):
        return pltpu.make_async_copy(
            table_hbm.at[idx_v.at[j]], rows_v.at[b], gsem.at[b])

    def writeback(j, b):
        start = pl.multiple_of(out_base // 2 + j * (CHUNK // 2), CHUNK // 2)
        return pltpu.make_async_copy(
            wb_v.at[b],
            out_hbm.at[pl.ds(start, CHUNK // 2)],
            osem.at[b])

    iota = lax.iota(jnp.int32, L)

    def extract(j, b):
        rows_b = rows_v.at[b]
        wb_b = wb_v.at[b]

        def row_body(i, carry):
            ji = jnp.full((L,), j, jnp.int32)
            si = jnp.full((L,), i, jnp.int32)
            di = jnp.full((L,), i // 2, jnp.int32)
            dbase = (i % 2) * DIM
            h64 = plsc.load_gather(offs_v, [ji, si])
            for k in range(DIM // L):
                col = (k * L) + iota
                v = plsc.load_gather(rows_b, [si, h64 + col])
                plsc.store_scatter(wb_b, [di, dbase + col], v)
            return carry

        lax.fori_loop(0, CHUNK, row_body, 0)

    for b in range(NBUF):
        prep_indices(b)
        gather(b, b).start()

    def group_body(g, carry):
        for b in range(NBUF):
            j = g * NBUF + b
            gather(j, b).wait()
            extract(j, b)
            writeback(j, b).start()
            jn = j + NBUF
            prep_indices(jn)
            writeback(j, b).wait()
            gather(jn, b).start()
        return carry

    lax.fori_loop(0, N_GROUPS - 1, group_body, 0)

    for b in range(NBUF):
        j = (N_GROUPS - 1) * NBUF + b
        gather(j, b).wait()
        extract(j, b)
        writeback(j, b).start()
    for b in range(NBUF):
        j = (N_GROUPS - 1) * NBUF + b
        writeback(j, b).wait()


@jax.jit
def _run(ids, offs_tiled, table_pairs):
    mesh = plsc.VectorSubcoreMesh(core_axis_name="c", subcore_axis_name="s")
    f = pl.kernel(
        _sc_gather,
        out_type=jax.ShapeDtypeStruct((N_ROWS // 2, PDIM), jnp.float32),
        mesh=mesh,
        scratch_types=[
            pltpu.VMEM((N_CHUNKS, CHUNK), jnp.int32),
            pltpu.VMEM((N_CHUNKS, CHUNK), jnp.int32),
            pltpu.VMEM((NBUF, CHUNK, PDIM), jnp.float32),
            pltpu.VMEM((NBUF, CHUNK // 2, PDIM), jnp.float32),
            pltpu.SemaphoreType.DMA((NBUF,)),
            pltpu.SemaphoreType.DMA((NBUF,)),
        ],
        compiler_params=pltpu.CompilerParams(needs_layout_passes=False),
    )
    return f(ids, offs_tiled, table_pairs)


def kernel(input_ids, offsets, table):
    ids = input_ids.astype(jnp.int32).reshape(NW, N_CHUNKS, CHUNK)
    offs_tiled = jnp.tile(offsets.astype(jnp.int32),
                          ROWS_PER_W // N_HEADS).reshape(N_CHUNKS, CHUNK)
    table_pairs = table.astype(jnp.float32).reshape(N_PAIRS, PDIM)
    out = _run(ids, offs_tiled, table_pairs)
    return out.reshape(BATCH, N_HEADS, DIM)
